# baseline (speedup 1.0000x reference)
import functools

import jax
import jax.numpy as jnp
from jax import lax
from jax.experimental import pallas as pl
from jax.experimental.pallas import tpu as pltpu
from jax.experimental.pallas import tpu_sc as plsc

NUM_ITEMS = 1000000
HIDDEN = 64
BATCH = 16384
NC, NS = 2, 16
NW = NC * NS
NR = 4
NWR = NW * NR
NH = 2
TAIL0 = 999936
NTC_E = TAIL0 // 128
CPR = NTC_E // NWR
CREM = NTC_E % NWR
HWIN = 4096
NCAP = 128
SPILL = 8
OUTB = BATCH + SPILL
ROW = 128


def _range_bounds(wr):
    col0 = wr * CPR + jnp.minimum(wr, CREM)
    ncols = jnp.where(wr < CREM, CPR + 1, CPR)
    scol = jnp.minimum(col0, (TAIL0 - NH * HWIN) // 128)
    return col0 * 128, (col0 + ncols) * 128, scol * 128


def _select_sc(idx):
    mesh = plsc.VectorSubcoreMesh(core_axis_name="c", subcore_axis_name="s")

    @functools.partial(
        pl.kernel,
        mesh=mesh,
        compiler_params=pltpu.CompilerParams(
            use_tc_tiling_on_sc=False, needs_layout_passes=False),
        out_type=(
            jax.ShapeDtypeStruct((NWR * NH * NCAP,), jnp.int32),
            jax.ShapeDtypeStruct((NWR * NH * NCAP,), jnp.int32),
            jax.ShapeDtypeStruct((NWR * NH * 8 + 16,), jnp.int32),
        ),
        scratch_types=[
            pltpu.VMEM((1024,), jnp.int32),
            pltpu.VMEM((NR * NH, NCAP), jnp.int32),
            pltpu.VMEM((NR * NH, NCAP), jnp.int32),
            pltpu.VMEM((16,), jnp.int32),
            pltpu.SemaphoreType.DMA,
        ],
    )
    def k(idx_hbm, l_out, b_out, n_out, scan_v, l_v, b_v, cnt_v, sem):
        wid = lax.axis_index("c") * NS + lax.axis_index("s")
        lanes = lax.iota(jnp.int32, 16)

        for rh in range(NR * NH):
            def prefill(k2, cnt, rh=rh):
                s16 = pl.ds(k2 * 16, 16)
                l_v[rh, s16] = jnp.zeros((16,), jnp.int32)
                b_v[rh, s16] = BATCH + (wid % SPILL) + jnp.zeros((16,), jnp.int32)
                return cnt

            lax.fori_loop(0, NCAP // 16, prefill, 0)

        bounds = []
        for r in range(NR):
            i_lo, i_hi, base = _range_bounds(wid * NR + r)
            hb1 = jnp.minimum(base + HWIN, TAIL0 - HWIN)
            bounds.append((i_lo, i_hi, base, hb1))

        def scan_piece(p2, cnts):
            pltpu.sync_copy(idx_hbm.at[pl.ds(p2 * 1024, 1024)], scan_v)

            def scan_vec(v, cnts2):
                ivec = scan_v[pl.ds(v * 16, 16)]
                bvec = lanes + (p2 * 1024 + v * 16)
                out = []
                for r in range(NR):
                    i_lo, i_hi, base, hb1 = bounds[r]
                    m = (ivec >= i_lo) & (ivec < i_hi)
                    m0 = m & (ivec < base + HWIN)
                    m1 = m & (ivec >= base + HWIN)
                    for (h, mh, wb) in ((0, m0, base), (1, m1, hb1)):
                        rh = r * NH + h
                        plsc.store_compressed(
                            l_v.at[rh].at[pl.ds(cnts2[rh], 16)],
                            ivec - wb, mask=mh)
                        plsc.store_compressed(
                            b_v.at[rh].at[pl.ds(cnts2[rh], 16)],
                            bvec, mask=mh)
                        out.append(cnts2[rh]
                                   + plsc.all_reduce_population_count(mh)[0])
                return tuple(out)

            return lax.fori_loop(0, 64, scan_vec, cnts)

        cnts = lax.fori_loop(0, 16, scan_piece, (0,) * (NR * NH))
        for rh in range(NR * NH):
            off = (wid * NR * NH + rh) * NCAP
            pltpu.sync_copy(l_v.at[rh], l_out.at[pl.ds(off, NCAP)])
            pltpu.sync_copy(b_v.at[rh], b_out.at[pl.ds(off, NCAP)])
            cnt_v[pl.ds(0, 16)] = jnp.broadcast_to(cnts[rh], (16,))
            pltpu.sync_copy(
                cnt_v.at[pl.ds(0, 8)],
                n_out.at[pl.ds((wid * NR * NH + rh) * 8, 8)])

    return k(idx)


def _stream_sc(l_list, b_list, n_list, pt, ct):
    mesh = plsc.VectorSubcoreMesh(core_axis_name="c", subcore_axis_name="s")

    @functools.partial(
        pl.kernel,
        mesh=mesh,
        compiler_params=pltpu.CompilerParams(needs_layout_passes=False),
        out_type=(
            jax.ShapeDtypeStruct((OUTB, ROW), jnp.float32),
            jax.ShapeDtypeStruct((OUTB, ROW), jnp.float32),
        ),
        scratch_types=[
            pltpu.VMEM((NR * NH * NCAP,), jnp.int32),
            pltpu.VMEM((NR * NH, NCAP), jnp.int32),
            pltpu.VMEM((80,), jnp.int32),
            pltpu.VMEM((8, HWIN), jnp.float32),
            pltpu.VMEM((8, HWIN), jnp.float32),
            pltpu.VMEM((NCAP, ROW), jnp.float32),
            pltpu.SemaphoreType.DMA,
            pltpu.SemaphoreType.DMA,
            pltpu.SemaphoreType.DMA,
        ],
    )
    def k(l_hbm, b_hbm, n_hbm, p_hbm, c_hbm, p_out, c_out,
          l_v, b_v, cnt_v, chA, chB, st_v, semA, semB, sem_s):
        wid = lax.axis_index("c") * NS + lax.axis_index("s")
        lanes = lax.iota(jnp.int32, 16)
        lo8 = lanes < 8
        rows8 = lanes & 7
        pair01 = jnp.where(lo8, 0, 1)

        loff = wid * NR * NH * NCAP
        pltpu.sync_copy(l_hbm.at[pl.ds(loff, NR * NH * NCAP)], l_v)
        for rh in range(NR * NH):
            pltpu.sync_copy(b_hbm.at[pl.ds(loff + rh * NCAP, NCAP)],
                            b_v.at[rh])
        pltpu.sync_copy(n_hbm.at[pl.ds(wid * NR * NH * 8, 80)], cnt_v)

        def wbase(rh):
            wr = wid * NR + rh // NH
            h = rh % NH
            col0 = wr * CPR + jnp.minimum(wr, CREM)
            base = jnp.minimum(col0, (TAIL0 - NH * HWIN) // 128) * 128
            hb1 = jnp.minimum(base + HWIN, TAIL0 - HWIN)
            return pl.multiple_of(jnp.where(h == 0, base, hb1), 128)

        NSTEP = NR * NH * (HIDDEN // 8)

        def make_table_phase(tab, out):
            def fire(s, buf, sem):
                rh = s // (HIDDEN // 8)
                a = pl.multiple_of((s % (HIDDEN // 8)) * 8, 8)
                return pltpu.async_copy(
                    tab.at[pl.ds(a, 8), pl.ds(wbase(rh), HWIN)], buf, sem)

            def drain(buf, sem):
                pltpu.make_async_copy(
                    tab.at[pl.ds(0, 8), pl.ds(0, HWIN)], buf, sem).wait()

            def extract_step(s, buf):
                rh = s // (HIDDEN // 8)
                a = s % (HIDDEN // 8)
                n = cnt_v[pl.ds(rh * 8, 16)][0]
                trips = (n + 15) >> 4

                def etr(k2, c2):
                    lvec = l_v[pl.ds(rh * NCAP + k2 * 16, 16)]
                    for j in range(0, 16, 2):
                        l0 = jnp.broadcast_to(lvec[j], (16,))
                        l1 = jnp.broadcast_to(lvec[j + 1], (16,))
                        cols = jnp.where(lo8, l0, l1)
                        vals = plsc.load_gather(buf, [rows8, cols])
                        posb = jnp.broadcast_to(k2 * 16 + j, (16,))
                        plsc.store_scatter(
                            st_v, [posb + pair01, a * 8 + rows8], vals)
                    return c2

                lax.fori_loop(0, trips, etr, 0)

                @pl.when(a == HIDDEN // 8 - 1)
                def _():
                    pltpu.async_copy(st_v, out.at[b_v.at[rh]], sem_s).wait()

            fire(0, chA, semA)
            fire(1, chB, semB)

            def body(u, carry):
                s0 = 2 * u
                drain(chA, semA)
                extract_step(s0, chA)
                fire(s0 + 2, chA, semA)
                drain(chB, semB)
                extract_step(s0 + 1, chB)
                fire(s0 + 3, chB, semB)
                return carry

            lax.fori_loop(0, NSTEP // 2 - 1, body, 0)
            drain(chA, semA)
            extract_step(NSTEP - 2, chA)
            drain(chB, semB)
            extract_step(NSTEP - 1, chB)

        make_table_phase(p_hbm, p_out)
        make_table_phase(c_hbm, c_out)

    return k(l_list, b_list, n_list, pt, ct)


def _rating_tc(pt, ct, W, b):
    blk = 4096

    def body(p_ref, c_ref, w_ref, b_ref, o_ref):
        s = jnp.sum((p_ref[...] + c_ref[...]) * w_ref[...], axis=0, keepdims=True)
        o_ref[...] = jax.nn.sigmoid(s + b_ref[...])

    return pl.pallas_call(
        body,
        grid=(BATCH // blk,),
        in_specs=[
            pl.BlockSpec((HIDDEN, blk), lambda i: (0, i)),
            pl.BlockSpec((HIDDEN, blk), lambda i: (0, i)),
            pl.BlockSpec((HIDDEN, 1), lambda i: (0, 0)),
            pl.BlockSpec((1, 1), lambda i: (0, 0)),
        ],
        out_specs=pl.BlockSpec((1, blk), lambda i: (0, i)),
        out_shape=jax.ShapeDtypeStruct((1, BATCH), jnp.float32),
    )(pt, ct, W.reshape(HIDDEN, 1), b.reshape(1, 1))


def kernel(item_indices, item_personality_table, item_commonality_table, W, b):
    idx = item_indices.astype(jnp.int32)
    l_list, b_list, n_list = _select_sc(idx)
    p_ext, c_ext = _stream_sc(
        l_list, b_list, n_list,
        item_personality_table.T, item_commonality_table.T)
    tmask = idx >= TAIL0
    tfix = jnp.where(tmask, idx - TAIL0, 0)
    ptail = jnp.take(item_personality_table[TAIL0:], tfix, axis=0)
    ctail = jnp.take(item_commonality_table[TAIL0:], tfix, axis=0)
    p = jnp.where(tmask[:, None], ptail, p_ext[:BATCH, :HIDDEN])
    c = jnp.where(tmask[:, None], ctail, c_ext[:BATCH, :HIDDEN])
    rating = _rating_tc(p.T, c.T, W, b).reshape(BATCH, 1)
    return (rating, p, c)


# speedup vs baseline: 1.0243x; 1.0243x over previous
import functools

import jax
import jax.numpy as jnp
from jax import lax
from jax.experimental import pallas as pl
from jax.experimental.pallas import tpu as pltpu
from jax.experimental.pallas import tpu_sc as plsc

NUM_ITEMS = 1000000
HIDDEN = 64
BATCH = 16384
NC, NS = 2, 16
NW = NC * NS
NR = 4
NWR = NW * NR
NH = 2
TAIL0 = 999936
NTC_E = TAIL0 // 128
CPR = NTC_E // NWR
CREM = NTC_E % NWR
HWIN = 4096
NCAP = 128
SPILL = 8
OUTB = BATCH + SPILL
ROW = 128


def _range_bounds(wr):
    col0 = wr * CPR + jnp.minimum(wr, CREM)
    ncols = jnp.where(wr < CREM, CPR + 1, CPR)
    scol = jnp.minimum(col0, (TAIL0 - NH * HWIN) // 128)
    return col0 * 128, (col0 + ncols) * 128, scol * 128


def _select_sc(idx):
    mesh = plsc.VectorSubcoreMesh(core_axis_name="c", subcore_axis_name="s")

    @functools.partial(
        pl.kernel,
        mesh=mesh,
        compiler_params=pltpu.CompilerParams(
            use_tc_tiling_on_sc=False, needs_layout_passes=False),
        out_type=(
            jax.ShapeDtypeStruct((NWR * NH * NCAP,), jnp.int32),
            jax.ShapeDtypeStruct((NWR * NH * NCAP,), jnp.int32),
            jax.ShapeDtypeStruct((NWR * NH * 8 + 16,), jnp.int32),
        ),
        scratch_types=[
            pltpu.VMEM((1024,), jnp.int32),
            pltpu.VMEM((NR * NH, NCAP), jnp.int32),
            pltpu.VMEM((NR * NH, NCAP), jnp.int32),
            pltpu.VMEM((16,), jnp.int32),
            pltpu.SemaphoreType.DMA,
        ],
    )
    def k(idx_hbm, l_out, b_out, n_out, scan_v, l_v, b_v, cnt_v, sem):
        wid = lax.axis_index("c") * NS + lax.axis_index("s")
        lanes = lax.iota(jnp.int32, 16)

        for rh in range(NR * NH):
            def prefill(k2, cnt, rh=rh):
                s16 = pl.ds(k2 * 16, 16)
                l_v[rh, s16] = jnp.zeros((16,), jnp.int32)
                b_v[rh, s16] = BATCH + (wid % SPILL) + jnp.zeros((16,), jnp.int32)
                return cnt

            lax.fori_loop(0, NCAP // 16, prefill, 0)

        bounds = []
        for r in range(NR):
            i_lo, i_hi, base = _range_bounds(wid * NR + r)
            hb1 = jnp.minimum(base + HWIN, TAIL0 - HWIN)
            bounds.append((i_lo, i_hi, base, hb1))

        def scan_piece(p2, cnts):
            pltpu.sync_copy(idx_hbm.at[pl.ds(p2 * 1024, 1024)], scan_v)

            def scan_vec(v, cnts2):
                ivec = scan_v[pl.ds(v * 16, 16)]
                bvec = lanes + (p2 * 1024 + v * 16)
                out = []
                for r in range(NR):
                    i_lo, i_hi, base, hb1 = bounds[r]
                    m = (ivec >= i_lo) & (ivec < i_hi)
                    m0 = m & (ivec < base + HWIN)
                    m1 = m & (ivec >= base + HWIN)
                    for (h, mh, wb) in ((0, m0, base), (1, m1, hb1)):
                        rh = r * NH + h
                        plsc.store_compressed(
                            l_v.at[rh].at[pl.ds(cnts2[rh], 16)],
                            ivec - wb, mask=mh)
                        plsc.store_compressed(
                            b_v.at[rh].at[pl.ds(cnts2[rh], 16)],
                            bvec, mask=mh)
                        out.append(cnts2[rh]
                                   + plsc.all_reduce_population_count(mh)[0])
                return tuple(out)

            return lax.fori_loop(0, 64, scan_vec, cnts)

        cnts = lax.fori_loop(0, 16, scan_piece, (0,) * (NR * NH))
        for rh in range(NR * NH):
            off = (wid * NR * NH + rh) * NCAP
            pltpu.sync_copy(l_v.at[rh], l_out.at[pl.ds(off, NCAP)])
            pltpu.sync_copy(b_v.at[rh], b_out.at[pl.ds(off, NCAP)])
            cnt_v[pl.ds(0, 16)] = jnp.broadcast_to(cnts[rh], (16,))
            pltpu.sync_copy(
                cnt_v.at[pl.ds(0, 8)],
                n_out.at[pl.ds((wid * NR * NH + rh) * 8, 8)])

    return k(idx)


def _stream_sc(l_list, b_list, n_list, pt, ct):
    mesh = plsc.VectorSubcoreMesh(core_axis_name="c", subcore_axis_name="s")

    @functools.partial(
        pl.kernel,
        mesh=mesh,
        compiler_params=pltpu.CompilerParams(needs_layout_passes=False),
        out_type=(
            jax.ShapeDtypeStruct((OUTB, ROW), jnp.float32),
            jax.ShapeDtypeStruct((OUTB, ROW), jnp.float32),
        ),
        scratch_types=[
            pltpu.VMEM((NR * NH * NCAP,), jnp.int32),
            pltpu.VMEM((NR * NH, NCAP), jnp.int32),
            pltpu.VMEM((80,), jnp.int32),
            pltpu.VMEM((8, HWIN), jnp.float32),
            pltpu.VMEM((8, HWIN), jnp.float32),
            pltpu.VMEM((NCAP, ROW), jnp.float32),
            pltpu.SemaphoreType.DMA,
            pltpu.SemaphoreType.DMA,
            pltpu.SemaphoreType.DMA,
        ],
    )
    def k(l_hbm, b_hbm, n_hbm, p_hbm, c_hbm, p_out, c_out,
          l_v, b_v, cnt_v, chA, chB, st_v, semA, semB, sem_s):
        wid = lax.axis_index("c") * NS + lax.axis_index("s")
        lanes = lax.iota(jnp.int32, 16)
        lo8 = lanes < 8
        rows8 = lanes & 7
        pair01 = jnp.where(lo8, 0, 1)

        loff = wid * NR * NH * NCAP
        pltpu.sync_copy(l_hbm.at[pl.ds(loff, NR * NH * NCAP)], l_v)
        for rh in range(NR * NH):
            pltpu.sync_copy(b_hbm.at[pl.ds(loff + rh * NCAP, NCAP)],
                            b_v.at[rh])
        pltpu.sync_copy(n_hbm.at[pl.ds(wid * NR * NH * 8, 80)], cnt_v)

        def round_body(r, carry0):
            wr = wid * NR + r
            _, _, base = _range_bounds(wr)
            base = pl.multiple_of(base, 128)
            hb1 = pl.multiple_of(
                jnp.minimum(base + HWIN, TAIL0 - HWIN), 128)

            for (tab, out) in ((p_hbm, p_out), (c_hbm, c_out)):
                steps = [(h, a) for h in range(NH) for a in range(HIDDEN // 8)]

                def fire(t):
                    h, a = steps[t]
                    buf, sem = (chA, semA) if t % 2 == 0 else (chB, semB)
                    wb = base if h == 0 else hb1
                    return pltpu.async_copy(
                        tab.at[pl.ds(a * 8, 8), pl.ds(wb, HWIN)], buf, sem)

                pending = fire(0)
                for t, (h, a) in enumerate(steps):
                    rh = r * NH + h
                    n = cnt_v[pl.ds(rh * 8, 16)][0]
                    trips = (n + 15) >> 4
                    nxt = fire(t + 1) if t + 1 < len(steps) else None
                    pending.wait()
                    pending = nxt
                    buf = chA if t % 2 == 0 else chB

                    def extract(k2, carry2, rh=rh, a=a, buf=buf):
                        lvec = l_v[pl.ds(rh * NCAP + k2 * 16, 16)]
                        for j in range(0, 16, 2):
                            l0 = jnp.broadcast_to(lvec[j], (16,))
                            l1 = jnp.broadcast_to(lvec[j + 1], (16,))
                            cols = jnp.where(lo8, l0, l1)
                            vals = plsc.load_gather(buf, [rows8, cols])
                            posb = jnp.broadcast_to(k2 * 16 + j, (16,))
                            plsc.store_scatter(
                                st_v, [posb + pair01, a * 8 + rows8], vals)
                        return carry2

                    lax.fori_loop(0, trips, extract, 0)
                    if a == HIDDEN // 8 - 1:
                        pltpu.async_copy(
                            st_v, out.at[b_v.at[rh]], sem_s).wait()
            return carry0

        lax.fori_loop(0, NR, round_body, 0)

    return k(l_list, b_list, n_list, pt, ct)


def _rating_tc(pt, ct, W, b):
    blk = 4096

    def body(p_ref, c_ref, w_ref, b_ref, o_ref):
        s = jnp.sum((p_ref[...] + c_ref[...]) * w_ref[...], axis=0, keepdims=True)
        o_ref[...] = jax.nn.sigmoid(s + b_ref[...])

    return pl.pallas_call(
        body,
        grid=(BATCH // blk,),
        in_specs=[
            pl.BlockSpec((HIDDEN, blk), lambda i: (0, i)),
            pl.BlockSpec((HIDDEN, blk), lambda i: (0, i)),
            pl.BlockSpec((HIDDEN, 1), lambda i: (0, 0)),
            pl.BlockSpec((1, 1), lambda i: (0, 0)),
        ],
        out_specs=pl.BlockSpec((1, blk), lambda i: (0, i)),
        out_shape=jax.ShapeDtypeStruct((1, BATCH), jnp.float32),
    )(pt, ct, W.reshape(HIDDEN, 1), b.reshape(1, 1))


def kernel(item_indices, item_personality_table, item_commonality_table, W, b):
    idx = item_indices.astype(jnp.int32)
    l_list, b_list, n_list = _select_sc(idx)
    p_ext, c_ext = _stream_sc(
        l_list, b_list, n_list,
        item_personality_table.T, item_commonality_table.T)
    tmask = idx >= TAIL0
    tfix = jnp.where(tmask, idx - TAIL0, 0)
    oh = (tfix[:, None] == jnp.arange(NUM_ITEMS - TAIL0)[None, :]
          ).astype(jnp.float32)
    ptail = oh @ item_personality_table[TAIL0:]
    ctail = oh @ item_commonality_table[TAIL0:]
    p = jnp.where(tmask[:, None], ptail, p_ext[:BATCH, :HIDDEN])
    c = jnp.where(tmask[:, None], ctail, c_ext[:BATCH, :HIDDEN])
    rating = _rating_tc(p.T, c.T, W, b).reshape(BATCH, 1)
    return (rating, p, c)


# speedup vs baseline: 1.0302x; 1.0058x over previous
import functools

import jax
import jax.numpy as jnp
from jax import lax
from jax.experimental import pallas as pl
from jax.experimental.pallas import tpu as pltpu
from jax.experimental.pallas import tpu_sc as plsc

NUM_ITEMS = 1000000
HIDDEN = 64
BATCH = 16384
NC, NS = 2, 16
NW = NC * NS
NR = 4
NWR = NW * NR
TAIL0 = 999936
NTC_E = TAIL0 // 128
CPR = NTC_E // NWR
CREM = NTC_E % NWR
HWIN = 4096
NCAP = 256
NSEG = NCAP // 128
SPILL = 8
OUTB = BATCH + SPILL
ROW = 128


def _range_bounds(wr):
    col0 = wr * CPR + jnp.minimum(wr, CREM)
    ncols = jnp.where(wr < CREM, CPR + 1, CPR)
    scol = jnp.minimum(col0, (TAIL0 - 2 * HWIN) // 128)
    return col0 * 128, (col0 + ncols) * 128, scol * 128


def _select_sc(idx):
    mesh = plsc.VectorSubcoreMesh(core_axis_name="c", subcore_axis_name="s")

    @functools.partial(
        pl.kernel,
        mesh=mesh,
        compiler_params=pltpu.CompilerParams(
            use_tc_tiling_on_sc=False, needs_layout_passes=False),
        out_type=(
            jax.ShapeDtypeStruct((NWR * NCAP,), jnp.int32),
            jax.ShapeDtypeStruct((NWR * NCAP,), jnp.int32),
        ),
        scratch_types=[
            pltpu.VMEM((1024,), jnp.int32),
            pltpu.VMEM((NR, NCAP), jnp.int32),
            pltpu.VMEM((NR, NCAP), jnp.int32),
            pltpu.SemaphoreType.DMA,
        ],
    )
    def k(idx_hbm, l_out, b_out, scan_v, l_v, b_v, sem):
        wid = lax.axis_index("c") * NS + lax.axis_index("s")
        lanes = lax.iota(jnp.int32, 16)

        for r in range(NR):
            def prefill(k2, cnt, r=r):
                s16 = pl.ds(k2 * 16, 16)
                l_v[r, s16] = jnp.zeros((16,), jnp.int32)
                b_v[r, s16] = BATCH + (wid % SPILL) + jnp.zeros((16,), jnp.int32)
                return cnt

            lax.fori_loop(0, NCAP // 16, prefill, 0)

        bounds = [_range_bounds(wid * NR + r) for r in range(NR)]

        def scan_piece(p2, cnts):
            pltpu.sync_copy(idx_hbm.at[pl.ds(p2 * 1024, 1024)], scan_v)

            def scan_vec(v, cnts2):
                ivec = scan_v[pl.ds(v * 16, 16)]
                bvec = lanes + (p2 * 1024 + v * 16)
                out = []
                for r in range(NR):
                    i_lo, i_hi, base = bounds[r]
                    m = (ivec >= i_lo) & (ivec < i_hi)
                    plsc.store_compressed(
                        l_v.at[r].at[pl.ds(cnts2[r], 16)], ivec - base, mask=m)
                    plsc.store_compressed(
                        b_v.at[r].at[pl.ds(cnts2[r], 16)], bvec, mask=m)
                    out.append(
                        cnts2[r] + plsc.all_reduce_population_count(m)[0])
                return tuple(out)

            return lax.fori_loop(0, 64, scan_vec, cnts)

        lax.fori_loop(0, 16, scan_piece, (0,) * NR)
        for r in range(NR):
            wr_off = (wid * NR + r) * NCAP
            pltpu.sync_copy(l_v.at[r], l_out.at[pl.ds(wr_off, NCAP)])
            pltpu.sync_copy(b_v.at[r], b_out.at[pl.ds(wr_off, NCAP)])

    return k(idx)


def _stream_sc(l_list, b_list, pt, ct):
    mesh = plsc.VectorSubcoreMesh(core_axis_name="c", subcore_axis_name="s")

    @functools.partial(
        pl.kernel,
        mesh=mesh,
        compiler_params=pltpu.CompilerParams(needs_layout_passes=False),
        out_type=(
            jax.ShapeDtypeStruct((OUTB, ROW), jnp.float32),
            jax.ShapeDtypeStruct((OUTB, ROW), jnp.float32),
        ),
        scratch_types=[
            pltpu.VMEM((NCAP,), jnp.int32),
            pltpu.VMEM((NSEG, 128), jnp.int32),
            pltpu.VMEM((8, HWIN), jnp.float32),
            pltpu.VMEM((8, HWIN), jnp.float32),
            pltpu.VMEM((NCAP + 8, ROW), jnp.float32),
            pltpu.SemaphoreType.DMA,
            pltpu.SemaphoreType.DMA,
            pltpu.SemaphoreType.DMA,
        ],
    )
    def k(l_hbm, b_hbm, p_hbm, c_hbm, p_out, c_out,
          l_v, b2_v, chA, chB, st_v, semA, semB, sem_s):
        wid = lax.axis_index("c") * NS + lax.axis_index("s")
        lanes = lax.iota(jnp.int32, 16)
        lo8 = lanes < 8
        rows8 = lanes & 7
        pair01 = jnp.where(lo8, 0, 1)

        def round_body(r, carry0):
            wr = wid * NR + r
            _, _, base = _range_bounds(wr)
            base = pl.multiple_of(base, 128)
            hb1 = pl.multiple_of(
                jnp.minimum(base + HWIN, TAIL0 - HWIN), 128)
            pltpu.sync_copy(l_hbm.at[pl.ds(wr * NCAP, NCAP)], l_v)
            for seg in range(NSEG):
                pltpu.sync_copy(
                    b_hbm.at[pl.ds(wr * NCAP + seg * 128, 128)],
                    b2_v.at[seg])

            for (tab, out) in ((p_hbm, p_out), (c_hbm, c_out)):
                steps = [(a, h) for a in range(HIDDEN // 8) for h in range(2)]

                def fire(t):
                    a, h = steps[t]
                    buf, sem = (chA, semA) if t % 2 == 0 else (chB, semB)
                    hb = base if h == 0 else hb1
                    return pltpu.async_copy(
                        tab.at[pl.ds(a * 8, 8), pl.ds(hb, HWIN)], buf, sem)

                pending = fire(0)
                for t, (a, h) in enumerate(steps):
                    nxt = fire(t + 1) if t + 1 < len(steps) else None
                    pending.wait()
                    pending = nxt
                    buf = chA if t % 2 == 0 else chB
                    hb_rel = (base if h == 0 else hb1) - base

                    def extract(k2, carry2, a=a, buf=buf, hb_rel=hb_rel):
                        lvec = l_v[pl.ds(k2 * 16, 16)]
                        for j in range(0, 16, 2):
                            l0 = jnp.broadcast_to(lvec[j], (16,))
                            l1 = jnp.broadcast_to(lvec[j + 1], (16,))
                            lsel = jnp.where(lo8, l0, l1) - hb_rel
                            valid = (lsel >= 0) & (lsel < HWIN)
                            cols = jnp.clip(lsel, 0, HWIN - 1)
                            vals = plsc.load_gather(buf, [rows8, cols])
                            posb = jnp.broadcast_to(k2 * 16 + j, (16,))
                            rowsel = jnp.where(valid, posb + pair01, NCAP)
                            plsc.store_scatter(
                                st_v, [rowsel, a * 8 + rows8], vals)
                        return carry2

                    lax.fori_loop(0, NCAP // 16, extract, 0)

                waits = []
                for seg in range(NSEG):
                    waits.append(pltpu.async_copy(
                        st_v.at[pl.ds(seg * 128, 128)],
                        out.at[b2_v.at[seg]], sem_s))
                for w in waits:
                    w.wait()
            return carry0

        lax.fori_loop(0, NR, round_body, 0)

    return k(l_list, b_list, pt, ct)


def _rating_tc(pt, ct, W, b):
    blk = 4096

    def body(p_ref, c_ref, w_ref, b_ref, o_ref):
        s = jnp.sum((p_ref[...] + c_ref[...]) * w_ref[...], axis=0, keepdims=True)
        o_ref[...] = jax.nn.sigmoid(s + b_ref[...])

    return pl.pallas_call(
        body,
        grid=(BATCH // blk,),
        in_specs=[
            pl.BlockSpec((HIDDEN, blk), lambda i: (0, i)),
            pl.BlockSpec((HIDDEN, blk), lambda i: (0, i)),
            pl.BlockSpec((HIDDEN, 1), lambda i: (0, 0)),
            pl.BlockSpec((1, 1), lambda i: (0, 0)),
        ],
        out_specs=pl.BlockSpec((1, blk), lambda i: (0, i)),
        out_shape=jax.ShapeDtypeStruct((1, BATCH), jnp.float32),
    )(pt, ct, W.reshape(HIDDEN, 1), b.reshape(1, 1))


def kernel(item_indices, item_personality_table, item_commonality_table, W, b):
    idx = item_indices.astype(jnp.int32)
    l_list, b_list = _select_sc(idx)
    p_ext, c_ext = _stream_sc(
        l_list, b_list, item_personality_table.T, item_commonality_table.T)
    tmask = idx >= TAIL0
    tfix = jnp.where(tmask, idx - TAIL0, 0)
    oh = (tfix[:, None] == jnp.arange(NUM_ITEMS - TAIL0)[None, :]
          ).astype(jnp.float32)
    ptail = oh @ item_personality_table[TAIL0:]
    ctail = oh @ item_commonality_table[TAIL0:]
    p = jnp.where(tmask[:, None], ptail, p_ext[:BATCH, :HIDDEN])
    c = jnp.where(tmask[:, None], ctail, c_ext[:BATCH, :HIDDEN])
    rating = _rating_tc(p.T, c.T, W, b).reshape(BATCH, 1)
    return (rating, p, c)


# speedup vs baseline: 1.1016x; 1.0693x over previous
import functools

import jax
import jax.numpy as jnp
from jax import lax
from jax.experimental import pallas as pl
from jax.experimental.pallas import tpu as pltpu
from jax.experimental.pallas import tpu_sc as plsc

NUM_ITEMS = 1000000
HIDDEN = 64
BATCH = 16384
NC, NS = 2, 16
NW = NC * NS
NR = 4
NWR = NW * NR
TAIL0 = 999936
NTC_E = TAIL0 // 128
CPR = NTC_E // NWR
CREM = NTC_E % NWR
HWIN = 4096
NCAP = 256
NSEG = NCAP // 128
SPILL = 8
OUTB = BATCH + SPILL
ROW = 128


def _range_bounds(wr):
    col0 = wr * CPR + jnp.minimum(wr, CREM)
    ncols = jnp.where(wr < CREM, CPR + 1, CPR)
    scol = jnp.minimum(col0, (TAIL0 - 2 * HWIN) // 128)
    return col0 * 128, (col0 + ncols) * 128, scol * 128


def _select_sc(idx):
    mesh = plsc.VectorSubcoreMesh(core_axis_name="c", subcore_axis_name="s")

    @functools.partial(
        pl.kernel,
        mesh=mesh,
        compiler_params=pltpu.CompilerParams(
            use_tc_tiling_on_sc=False, needs_layout_passes=False),
        out_type=(
            jax.ShapeDtypeStruct((NWR * NCAP,), jnp.int32),
            jax.ShapeDtypeStruct((NWR * NCAP,), jnp.int32),
        ),
        scratch_types=[
            pltpu.VMEM((1024,), jnp.int32),
            pltpu.VMEM((NR, NCAP), jnp.int32),
            pltpu.VMEM((NR, NCAP), jnp.int32),
            pltpu.SemaphoreType.DMA,
        ],
    )
    def k(idx_hbm, l_out, b_out, scan_v, l_v, b_v, sem):
        wid = lax.axis_index("c") * NS + lax.axis_index("s")
        lanes = lax.iota(jnp.int32, 16)

        for r in range(NR):
            def prefill(k2, cnt, r=r):
                s16 = pl.ds(k2 * 16, 16)
                l_v[r, s16] = jnp.zeros((16,), jnp.int32)
                b_v[r, s16] = BATCH + (wid % SPILL) + jnp.zeros((16,), jnp.int32)
                return cnt

            lax.fori_loop(0, NCAP // 16, prefill, 0)

        bounds = [_range_bounds(wid * NR + r) for r in range(NR)]

        def scan_piece(p2, cnts):
            pltpu.sync_copy(idx_hbm.at[pl.ds(p2 * 1024, 1024)], scan_v)

            def scan_vec(v, cnts2):
                ivec = scan_v[pl.ds(v * 16, 16)]
                bvec = lanes + (p2 * 1024 + v * 16)
                out = []
                for r in range(NR):
                    i_lo, i_hi, base = bounds[r]
                    m = (ivec >= i_lo) & (ivec < i_hi)
                    plsc.store_compressed(
                        l_v.at[r].at[pl.ds(cnts2[r], 16)], ivec - base, mask=m)
                    plsc.store_compressed(
                        b_v.at[r].at[pl.ds(cnts2[r], 16)], bvec, mask=m)
                    out.append(
                        cnts2[r] + plsc.all_reduce_population_count(m)[0])
                return tuple(out)

            return lax.fori_loop(0, 64, scan_vec, cnts)

        lax.fori_loop(0, 16, scan_piece, (0,) * NR)
        for r in range(NR):
            wr_off = (wid * NR + r) * NCAP
            pltpu.sync_copy(l_v.at[r], l_out.at[pl.ds(wr_off, NCAP)])
            pltpu.sync_copy(b_v.at[r], b_out.at[pl.ds(wr_off, NCAP)])

    return k(idx)


def _stream_sc(l_list, b_list, pt, ct):
    mesh = plsc.VectorSubcoreMesh(core_axis_name="c", subcore_axis_name="s")

    @functools.partial(
        pl.kernel,
        mesh=mesh,
        compiler_params=pltpu.CompilerParams(needs_layout_passes=False),
        out_type=(
            jax.ShapeDtypeStruct((OUTB, ROW), jnp.float32),
            jax.ShapeDtypeStruct((OUTB, ROW), jnp.float32),
        ),
        scratch_types=[
            pltpu.VMEM((NCAP,), jnp.int32),
            pltpu.VMEM((NSEG, 128), jnp.int32),
            pltpu.VMEM((8, HWIN), jnp.float32),
            pltpu.VMEM((8, HWIN), jnp.float32),
            pltpu.VMEM((NCAP + 8, ROW), jnp.float32),
            pltpu.SemaphoreType.DMA,
            pltpu.SemaphoreType.DMA,
            pltpu.SemaphoreType.DMA,
        ],
    )
    def k(l_hbm, b_hbm, p_hbm, c_hbm, p_out, c_out,
          l_v, b2_v, chA, chB, st_v, semA, semB, sem_s):
        wid = lax.axis_index("c") * NS + lax.axis_index("s")
        lanes = lax.iota(jnp.int32, 16)
        lo8 = lanes < 8
        rows8 = lanes & 7
        pair01 = jnp.where(lo8, 0, 1)

        def round_body(r, carry0):
            wr = wid * NR + r
            _, _, base = _range_bounds(wr)
            base = pl.multiple_of(base, 128)
            hb1 = pl.multiple_of(
                jnp.minimum(base + HWIN, TAIL0 - HWIN), 128)
            pltpu.sync_copy(l_hbm.at[pl.ds(wr * NCAP, NCAP)], l_v)
            for seg in range(NSEG):
                pltpu.sync_copy(
                    b_hbm.at[pl.ds(wr * NCAP + seg * 128, 128)],
                    b2_v.at[seg])

            for (tab, out) in ((p_hbm, p_out), (c_hbm, c_out)):
                steps = [(a, h) for a in range(HIDDEN // 8) for h in range(2)]

                def fire(t):
                    a, h = steps[t]
                    buf, sem = (chA, semA) if t % 2 == 0 else (chB, semB)
                    hb = base if h == 0 else hb1
                    return pltpu.async_copy(
                        tab.at[pl.ds(a * 8, 8), pl.ds(hb, HWIN)], buf, sem)

                pending = fire(0)
                for t, (a, h) in enumerate(steps):
                    nxt = fire(t + 1) if t + 1 < len(steps) else None
                    pending.wait()
                    pending = nxt
                    buf = chA if t % 2 == 0 else chB
                    hb_rel = (base if h == 0 else hb1) - base

                    def extract(k2, carry2, a=a, buf=buf, hb_rel=hb_rel):
                        lvec = l_v[pl.ds(k2 * 16, 16)]
                        for j in range(0, 16, 2):
                            l0 = jnp.broadcast_to(lvec[j], (16,))
                            l1 = jnp.broadcast_to(lvec[j + 1], (16,))
                            lsel = jnp.where(lo8, l0, l1) - hb_rel
                            valid = (lsel >= 0) & (lsel < HWIN)
                            cols = jnp.clip(lsel, 0, HWIN - 1)
                            vals = plsc.load_gather(buf, [rows8, cols])
                            posb = jnp.broadcast_to(k2 * 16 + j, (16,))
                            rowsel = jnp.where(valid, posb + pair01, NCAP)
                            plsc.store_scatter(
                                st_v, [rowsel, a * 8 + rows8], vals)
                        return carry2

                    lax.fori_loop(0, NCAP // 16, extract, 0)

                waits = []
                for seg in range(NSEG):
                    waits.append(pltpu.async_copy(
                        st_v.at[pl.ds(seg * 128, 128)],
                        out.at[b2_v.at[seg]], sem_s))
                for w in waits:
                    w.wait()
            return carry0

        lax.fori_loop(0, NR, round_body, 0)

    return k(l_list, b_list, pt, ct)


def _rating_tc(pt, ct, W, b):
    blk = 4096

    def body(p_ref, c_ref, w_ref, b_ref, o_ref):
        s = jnp.sum((p_ref[...] + c_ref[...]) * w_ref[...], axis=0, keepdims=True)
        o_ref[...] = jax.nn.sigmoid(s + b_ref[...])

    return pl.pallas_call(
        body,
        grid=(BATCH // blk,),
        in_specs=[
            pl.BlockSpec((HIDDEN, blk), lambda i: (0, i)),
            pl.BlockSpec((HIDDEN, blk), lambda i: (0, i)),
            pl.BlockSpec((HIDDEN, 1), lambda i: (0, 0)),
            pl.BlockSpec((1, 1), lambda i: (0, 0)),
        ],
        out_specs=pl.BlockSpec((1, blk), lambda i: (0, i)),
        out_shape=jax.ShapeDtypeStruct((1, BATCH), jnp.float32),
    )(pt, ct, W.reshape(HIDDEN, 1), b.reshape(1, 1))


def kernel(item_indices, item_personality_table, item_commonality_table, W, b):
    idx = item_indices.astype(jnp.int32)
    l_list, b_list = _select_sc(idx)
    p_ext, c_ext = _stream_sc(
        l_list, b_list, item_personality_table.T, item_commonality_table.T)
    tmask = idx >= TAIL0
    tfix = jnp.where(tmask, idx - TAIL0, 0)
    ptail = jnp.take(item_personality_table[TAIL0:], tfix, axis=0)
    ctail = jnp.take(item_commonality_table[TAIL0:], tfix, axis=0)
    p = jnp.where(tmask[:, None], ptail, p_ext[:BATCH, :HIDDEN])
    c = jnp.where(tmask[:, None], ctail, c_ext[:BATCH, :HIDDEN])
    rating = _rating_tc(p.T, c.T, W, b).reshape(BATCH, 1)
    return (rating, p, c)
